# baseline (speedup 1.0000x reference)
import jax
import jax.numpy as jnp
from jax.experimental import pallas as pl
from jax.experimental.pallas import tpu as pltpu

B = 2048
DIN = 1024
HID = 1024
NC = 3
E = 16
ED = 128
PH = 64
DENH = HID + 2 * PH
KWIN = 20
DECAY = 0.8
DELTA0 = 7.0
TB = 512
BF = jnp.bfloat16
F32 = jnp.float32


def _dot(a, b):
    return jnp.dot(a, b, preferred_element_type=F32)


def _k1(x_ref, win_ref, bin_ref, wr_ref, br_ref, enh_ref, gate_ref, win_bf):
    i = pl.program_id(0)

    @pl.when(i == 0)
    def _():
        win_bf[...] = win_ref[...].astype(BF)

    proj = _dot(x_ref[...].astype(BF), win_bf[...]) + bin_ref[...]
    xm = jnp.mean(proj, axis=1, keepdims=True)
    harm = jax.lax.broadcasted_iota(jnp.int32, (1, PH), 1).astype(F32) + 1.0
    phase = DELTA0 * harm * xm
    temporal = jnp.concatenate([jnp.cos(phase), jnp.sin(phase)], axis=1)
    enh = jnp.concatenate([proj, temporal], axis=1).astype(BF)
    enh_ref[...] = enh
    logits = _dot(enh, wr_ref[...].astype(BF)) + br_ref[...]
    col = jax.lax.broadcasted_iota(jnp.int32, (TB, E), 1)
    m1 = jnp.max(logits, axis=1, keepdims=True)
    i1 = jnp.min(jnp.where(logits == m1, col, E), axis=1, keepdims=True)
    l2 = jnp.where(col == i1, -jnp.inf, logits)
    m2 = jnp.max(l2, axis=1, keepdims=True)
    i2 = jnp.min(jnp.where(l2 == m2, col, E), axis=1, keepdims=True)
    g1 = 1.0 / (1.0 + jnp.exp(m2 - m1))
    g2 = 1.0 - g1
    gate_ref[...] = jnp.where(col == i1, g1, 0.0) + jnp.where(col == i2, g2, 0.0)


def _gains_from_c0(c0):
    bits = jax.lax.bitcast_convert_type(jnp.abs(c0), jnp.int32)

    def body(_, lohi):
        lo, hi = lohi
        mid = lo + (hi - lo + 1) // 2
        ge = jnp.sum((bits >= mid).astype(jnp.int32)) >= 100
        return (jnp.where(ge, mid, lo), jnp.where(ge, hi, mid - 1))

    lo, _ = jax.lax.fori_loop(0, 31, body, (jnp.int32(0), jnp.int32(0x7F800000)))
    sel = (bits >= lo).astype(F32)
    c = sel
    k = 1
    while k < HID:
        c = c + jnp.concatenate([jnp.zeros((1, k), F32), c[:, :-k]], axis=1)
        k *= 2
    winner = (sel > 0.0) & (c <= KWIN + 0.5)
    return 1.0 + DECAY * DECAY * winner.astype(F32)


TB3 = B // 4


def _k3s(x_ref, win_ref, bin_ref, wr_ref, br_ref, w1_ref, b1s_ref, b1e_ref,
         w2_ref, b2_ref, out_ref, gains_ref, win_bf, w1s, w2s):
    i = pl.program_id(0)

    @pl.when(i == 0)
    def _():
        win_bf[...] = win_ref[...].astype(BF)
        for e in range(E):
            w1s[:, e * ED:(e + 1) * ED] = w1_ref[e].astype(BF)
            w2s[e * ED:(e + 1) * ED, :] = w2_ref[e].astype(BF)

    proj = _dot(x_ref[...].astype(BF), win_bf[...]) + bin_ref[...]
    xm = jnp.mean(proj, axis=1, keepdims=True)
    harm = jax.lax.broadcasted_iota(jnp.int32, (1, PH), 1).astype(F32) + 1.0
    phase = DELTA0 * harm * xm
    temporal = jnp.concatenate([jnp.cos(phase), jnp.sin(phase)], axis=1)
    enh = jnp.concatenate([proj, temporal], axis=1).astype(BF)
    logits = _dot(enh, wr_ref[...].astype(BF)) + br_ref[...]
    col = jax.lax.broadcasted_iota(jnp.int32, (TB3, E), 1)
    m1 = jnp.max(logits, axis=1, keepdims=True)
    i1 = jnp.min(jnp.where(logits == m1, col, E), axis=1, keepdims=True)
    l2 = jnp.where(col == i1, -jnp.inf, logits)
    m2 = jnp.max(l2, axis=1, keepdims=True)
    i2 = jnp.min(jnp.where(l2 == m2, col, E), axis=1, keepdims=True)
    g1 = 1.0 / (1.0 + jnp.exp(m2 - m1))
    g2 = 1.0 - g1
    gate = jnp.where(col == i1, g1, 0.0) + jnp.where(col == i2, g2, 0.0)
    gate_bf = gate.astype(BF)
    h = jnp.maximum(_dot(enh, w1s[...]) + b1s_ref[...], 0.0)
    parts = [
        (h[:, e * ED:(e + 1) * ED] * gate[:, e:e + 1]).astype(BF)
        for e in range(E)
    ]
    s = jnp.concatenate(parts, axis=1)
    bias2 = _dot(gate_bf, b2_ref[...].astype(BF))
    out_ref[...] = _dot(s, w2s[...]) + bias2

    @pl.when(i == 0)
    def _():
        grow = gate[0:1, :]
        colg = jax.lax.broadcasted_iota(jnp.int32, (1, E), 1)
        nz = grow > 0.0
        ia = jnp.min(jnp.where(nz, colg, E))
        ib = jnp.max(jnp.where(nz, colg, -1))
        enh0 = enh[0:8, :]

        def pe_for(ie):
            w1e = w1_ref[pl.ds(ie, 1)][0].astype(BF)
            b1e = b1e_ref[pl.ds(ie, 1)][0]
            h1 = jnp.maximum(_dot(enh0, w1e) + b1e, 0.0)
            w2e = w2_ref[pl.ds(ie, 1)][0].astype(BF)
            b2e = b2_ref[pl.ds(ie, 1), :]
            pe = _dot(h1.astype(BF), w2e) + b2e
            gv = jnp.sum(jnp.where(colg == ie, grow, 0.0))
            return gv.astype(BF).astype(F32), pe.astype(BF).astype(F32)

        ga, pea = pe_for(ia)
        gb, peb = pe_for(ib)
        ctx0 = ga * pea + (ib != ia).astype(F32) * gb * peb
        out_ref[0:1, :] = ctx0[0:1, :]
        gains_ref[...] = _gains_from_c0(out_ref[0:1, :])


def _k5(ctx_ref, gains_ref, wh1, bh1, wh2, bh2, wh3, bh3, wo, bo, out_ref,
        wh1_bf, wh2_bf, wh3_bf):
    i = pl.program_id(0)

    @pl.when(i == 0)
    def _():
        wh1_bf[...] = wh1[...].astype(BF)
        wh2_bf[...] = wh2[...].astype(BF)
        wh3_bf[...] = wh3[...].astype(BF)

    att = (ctx_ref[...] * gains_ref[...]).astype(BF)
    h = jnp.maximum(_dot(att, wh1_bf[...]) + bh1[...], 0.0)
    h = jnp.maximum(_dot(h.astype(BF), wh2_bf[...]) + bh2[...], 0.0)
    h = jnp.maximum(_dot(h.astype(BF), wh3_bf[...]) + bh3[...], 0.0)
    out_ref[...] = _dot(h.astype(BF), wo[...].astype(BF)) + bo[...]


def kernel(x, W_in, b_in, W_r, b_r, W1, b1, W2, b2, Wh1, bh1, Wh2, bh2, Wh3, bh3, W_out, b_out):
    b_in2 = b_in.reshape(1, HID)
    b_r2 = b_r.reshape(1, E)
    bh1_2 = bh1.reshape(1, HID)
    bh2_2 = bh2.reshape(1, HID)
    bh3_2 = bh3.reshape(1, HID)
    b_out2 = b_out.reshape(1, NC)

    nt = B // TB
    context, gains = pl.pallas_call(
        _k3s,
        grid=(B // TB3,),
        in_specs=[
            pl.BlockSpec((TB3, DIN), lambda i: (i, 0)),
            pl.BlockSpec((DIN, HID), lambda i: (0, 0)),
            pl.BlockSpec((1, HID), lambda i: (0, 0)),
            pl.BlockSpec((DENH, E), lambda i: (0, 0)),
            pl.BlockSpec((1, E), lambda i: (0, 0)),
            pl.BlockSpec((E, DENH, ED), lambda i: (0, 0, 0)),
            pl.BlockSpec((1, E * ED), lambda i: (0, 0)),
            pl.BlockSpec((E, 1, ED), lambda i: (0, 0, 0)),
            pl.BlockSpec((E, ED, HID), lambda i: (0, 0, 0)),
            pl.BlockSpec((E, HID), lambda i: (0, 0)),
        ],
        out_specs=[
            pl.BlockSpec((TB3, HID), lambda i: (i, 0)),
            pl.BlockSpec((1, HID), lambda i: (0, 0)),
        ],
        out_shape=[
            jax.ShapeDtypeStruct((B, HID), F32),
            jax.ShapeDtypeStruct((1, HID), F32),
        ],
        scratch_shapes=[
            pltpu.VMEM((DIN, HID), BF),
            pltpu.VMEM((DENH, E * ED), BF),
            pltpu.VMEM((E * ED, HID), BF),
        ],
    )(x, W_in, b_in2, W_r, b_r2, W1, b1.reshape(1, E * ED),
      b1.reshape(E, 1, ED), W2, b2)

    out = pl.pallas_call(
        _k5,
        grid=(nt,),
        in_specs=[
            pl.BlockSpec((TB, HID), lambda i: (i, 0)),
            pl.BlockSpec((1, HID), lambda i: (0, 0)),
            pl.BlockSpec((HID, HID), lambda i: (0, 0)),
            pl.BlockSpec((1, HID), lambda i: (0, 0)),
            pl.BlockSpec((HID, HID), lambda i: (0, 0)),
            pl.BlockSpec((1, HID), lambda i: (0, 0)),
            pl.BlockSpec((HID, HID), lambda i: (0, 0)),
            pl.BlockSpec((1, HID), lambda i: (0, 0)),
            pl.BlockSpec((HID, NC), lambda i: (0, 0)),
            pl.BlockSpec((1, NC), lambda i: (0, 0)),
        ],
        out_specs=pl.BlockSpec((TB, NC), lambda i: (i, 0)),
        out_shape=jax.ShapeDtypeStruct((B, NC), F32),
        scratch_shapes=[
            pltpu.VMEM((HID, HID), BF),
            pltpu.VMEM((HID, HID), BF),
            pltpu.VMEM((HID, HID), BF),
        ],
    )(context, gains, Wh1, bh1_2, Wh2, bh2_2, Wh3, bh3_2, W_out, b_out2)
    return out


# speedup vs baseline: 1.0799x; 1.0799x over previous
import jax
import jax.numpy as jnp
from jax.experimental import pallas as pl
from jax.experimental.pallas import tpu as pltpu

B = 2048
DIN = 1024
HID = 1024
NC = 3
E = 16
ED = 128
PH = 64
DENH = HID + 2 * PH
KWIN = 20
DECAY = 0.8
DELTA0 = 7.0
TB = 512
BF = jnp.bfloat16
F32 = jnp.float32


def _dot(a, b):
    return jnp.dot(a, b, preferred_element_type=F32)


def _k1(x_ref, win_ref, bin_ref, wr_ref, br_ref, enh_ref, gate_ref, win_bf):
    i = pl.program_id(0)

    @pl.when(i == 0)
    def _():
        win_bf[...] = win_ref[...].astype(BF)

    proj = _dot(x_ref[...].astype(BF), win_bf[...]) + bin_ref[...]
    xm = jnp.mean(proj, axis=1, keepdims=True)
    harm = jax.lax.broadcasted_iota(jnp.int32, (1, PH), 1).astype(F32) + 1.0
    phase = DELTA0 * harm * xm
    temporal = jnp.concatenate([jnp.cos(phase), jnp.sin(phase)], axis=1)
    enh = jnp.concatenate([proj, temporal], axis=1).astype(BF)
    enh_ref[...] = enh
    logits = _dot(enh, wr_ref[...].astype(BF)) + br_ref[...]
    col = jax.lax.broadcasted_iota(jnp.int32, (TB, E), 1)
    m1 = jnp.max(logits, axis=1, keepdims=True)
    i1 = jnp.min(jnp.where(logits == m1, col, E), axis=1, keepdims=True)
    l2 = jnp.where(col == i1, -jnp.inf, logits)
    m2 = jnp.max(l2, axis=1, keepdims=True)
    i2 = jnp.min(jnp.where(l2 == m2, col, E), axis=1, keepdims=True)
    g1 = 1.0 / (1.0 + jnp.exp(m2 - m1))
    g2 = 1.0 - g1
    gate_ref[...] = jnp.where(col == i1, g1, 0.0) + jnp.where(col == i2, g2, 0.0)


def _gains_from_c0(c0):
    bits = jax.lax.bitcast_convert_type(jnp.abs(c0), jnp.int32)

    def body(_, lohi):
        lo, hi = lohi
        mid = lo + (hi - lo + 1) // 2
        ge = jnp.sum((bits >= mid).astype(jnp.int32)) >= 100
        return (jnp.where(ge, mid, lo), jnp.where(ge, hi, mid - 1))

    lo, _ = jax.lax.fori_loop(0, 31, body, (jnp.int32(0), jnp.int32(0x7F800000)))
    sel = (bits >= lo).astype(F32)
    c = sel
    k = 1
    while k < HID:
        c = c + jnp.concatenate([jnp.zeros((1, k), F32), c[:, :-k]], axis=1)
        k *= 2
    winner = (sel > 0.0) & (c <= KWIN + 0.5)
    return 1.0 + DECAY * DECAY * winner.astype(F32)


TB3 = B // 2


def _k3s(enh_ref, w1_ref, b1s_ref, b1e_ref, w2_ref, b2_ref, gate_ref,
         out_ref, gains_ref, w1s, w2s):
    i = pl.program_id(0)

    @pl.when(i == 0)
    def _():
        for e in range(E):
            w1s[:, e * ED:(e + 1) * ED] = w1_ref[e].astype(BF)
            w2s[e * ED:(e + 1) * ED, :] = w2_ref[e].astype(BF)

    gate = gate_ref[...]
    gate_bf = gate.astype(BF)
    h = jnp.maximum(_dot(enh_ref[...], w1s[...]) + b1s_ref[...], 0.0)
    parts = [
        (h[:, e * ED:(e + 1) * ED] * gate[:, e:e + 1]).astype(BF)
        for e in range(E)
    ]
    s = jnp.concatenate(parts, axis=1)
    bias2 = _dot(gate_bf, b2_ref[...].astype(BF))
    out_ref[...] = _dot(s, w2s[...]) + bias2

    @pl.when(i == 0)
    def _():
        grow = gate[0:1, :]
        colg = jax.lax.broadcasted_iota(jnp.int32, (1, E), 1)
        nz = grow > 0.0
        ia = jnp.min(jnp.where(nz, colg, E))
        ib = jnp.max(jnp.where(nz, colg, -1))
        enh0 = enh_ref[0:8, :]

        def pe_for(ie):
            w1e = w1_ref[pl.ds(ie, 1)][0].astype(BF)
            b1e = b1e_ref[pl.ds(ie, 1)][0]
            h1 = jnp.maximum(_dot(enh0, w1e) + b1e, 0.0)
            w2e = w2_ref[pl.ds(ie, 1)][0].astype(BF)
            b2e = b2_ref[pl.ds(ie, 1), :]
            pe = _dot(h1.astype(BF), w2e) + b2e
            gv = jnp.sum(jnp.where(colg == ie, grow, 0.0))
            return gv.astype(BF).astype(F32), pe.astype(BF).astype(F32)

        ga, pea = pe_for(ia)
        gb, peb = pe_for(ib)
        ctx0 = ga * pea + (ib != ia).astype(F32) * gb * peb
        out_ref[0:1, :] = ctx0[0:1, :]
        gains_ref[...] = _gains_from_c0(out_ref[0:1, :])


def _k5(ctx_ref, gains_ref, wh1, bh1, wh2, bh2, wh3, bh3, wo, bo, out_ref,
        wh1_bf, wh2_bf, wh3_bf):
    i = pl.program_id(0)

    @pl.when(i == 0)
    def _():
        wh1_bf[...] = wh1[...].astype(BF)
        wh2_bf[...] = wh2[...].astype(BF)
        wh3_bf[...] = wh3[...].astype(BF)

    att = (ctx_ref[...] * gains_ref[...]).astype(BF)
    h = jnp.maximum(_dot(att, wh1_bf[...]) + bh1[...], 0.0)
    h = jnp.maximum(_dot(h.astype(BF), wh2_bf[...]) + bh2[...], 0.0)
    h = jnp.maximum(_dot(h.astype(BF), wh3_bf[...]) + bh3[...], 0.0)
    out_ref[...] = _dot(h.astype(BF), wo[...].astype(BF)) + bo[...]


def kernel(x, W_in, b_in, W_r, b_r, W1, b1, W2, b2, Wh1, bh1, Wh2, bh2, Wh3, bh3, W_out, b_out):
    b_in2 = b_in.reshape(1, HID)
    b_r2 = b_r.reshape(1, E)
    bh1_2 = bh1.reshape(1, HID)
    bh2_2 = bh2.reshape(1, HID)
    bh3_2 = bh3.reshape(1, HID)
    b_out2 = b_out.reshape(1, NC)

    nt = B // TB
    enh, gate_full = pl.pallas_call(
        _k1,
        grid=(nt,),
        in_specs=[
            pl.BlockSpec((TB, DIN), lambda i: (i, 0)),
            pl.BlockSpec((DIN, HID), lambda i: (0, 0)),
            pl.BlockSpec((1, HID), lambda i: (0, 0)),
            pl.BlockSpec((DENH, E), lambda i: (0, 0)),
            pl.BlockSpec((1, E), lambda i: (0, 0)),
        ],
        out_specs=[
            pl.BlockSpec((TB, DENH), lambda i: (i, 0)),
            pl.BlockSpec((TB, E), lambda i: (i, 0)),
        ],
        out_shape=[
            jax.ShapeDtypeStruct((B, DENH), BF),
            jax.ShapeDtypeStruct((B, E), F32),
        ],
        scratch_shapes=[pltpu.VMEM((DIN, HID), BF)],
    )(x, W_in, b_in2, W_r, b_r2)

    context, gains = pl.pallas_call(
        _k3s,
        grid=(B // TB3,),
        in_specs=[
            pl.BlockSpec((TB3, DENH), lambda i: (i, 0)),
            pl.BlockSpec((E, DENH, ED), lambda i: (0, 0, 0)),
            pl.BlockSpec((1, E * ED), lambda i: (0, 0)),
            pl.BlockSpec((E, 1, ED), lambda i: (0, 0, 0)),
            pl.BlockSpec((E, ED, HID), lambda i: (0, 0, 0)),
            pl.BlockSpec((E, HID), lambda i: (0, 0)),
            pl.BlockSpec((TB3, E), lambda i: (i, 0)),
        ],
        out_specs=[
            pl.BlockSpec((TB3, HID), lambda i: (i, 0)),
            pl.BlockSpec((1, HID), lambda i: (0, 0)),
        ],
        out_shape=[
            jax.ShapeDtypeStruct((B, HID), F32),
            jax.ShapeDtypeStruct((1, HID), F32),
        ],
        scratch_shapes=[
            pltpu.VMEM((DENH, E * ED), BF),
            pltpu.VMEM((E * ED, HID), BF),
        ],
    )(enh, W1, b1.reshape(1, E * ED), b1.reshape(E, 1, ED), W2, b2, gate_full)

    out = pl.pallas_call(
        _k5,
        grid=(nt,),
        in_specs=[
            pl.BlockSpec((TB, HID), lambda i: (i, 0)),
            pl.BlockSpec((1, HID), lambda i: (0, 0)),
            pl.BlockSpec((HID, HID), lambda i: (0, 0)),
            pl.BlockSpec((1, HID), lambda i: (0, 0)),
            pl.BlockSpec((HID, HID), lambda i: (0, 0)),
            pl.BlockSpec((1, HID), lambda i: (0, 0)),
            pl.BlockSpec((HID, HID), lambda i: (0, 0)),
            pl.BlockSpec((1, HID), lambda i: (0, 0)),
            pl.BlockSpec((HID, NC), lambda i: (0, 0)),
            pl.BlockSpec((1, NC), lambda i: (0, 0)),
        ],
        out_specs=pl.BlockSpec((TB, NC), lambda i: (i, 0)),
        out_shape=jax.ShapeDtypeStruct((B, NC), F32),
        scratch_shapes=[
            pltpu.VMEM((HID, HID), BF),
            pltpu.VMEM((HID, HID), BF),
            pltpu.VMEM((HID, HID), BF),
        ],
    )(context, gains, Wh1, bh1_2, Wh2, bh2_2, Wh3, bh3_2, W_out, b_out2)
    return out
